# baseline (speedup 1.0000x reference)
import jax
import jax.numpy as jnp
from jax import lax
from jax.experimental import pallas as pl
from jax.experimental.pallas import tpu as pltpu
from jax.experimental.pallas import tpu_sc as plsc

N = 10000
E = 320000
D = 128

N_PAD = 10240
DUMMY_ROW = N
CHUNK = 128
N_WORKERS = 32
K = 8
G = 10
CHUNKS_PER_TILE = G * K
EDGES_PER_TILE = CHUNKS_PER_TILE * CHUNK
E_PAD = N_WORKERS * EDGES_PER_TILE
ROWS_PER_TILE = N_PAD // 16


def _make_sc_agg(with_cnt: bool):
    mesh = plsc.VectorSubcoreMesh(core_axis_name="c", subcore_axis_name="s")
    out_type = [jax.ShapeDtypeStruct((2, N_PAD, D), jnp.float32)]
    scratch = [
        pltpu.VMEM_SHARED((N_PAD, D), jnp.float32),
        pltpu.VMEM((2, K, CHUNK), jnp.int32),
        pltpu.VMEM((2, K, CHUNK), jnp.int32),
        pltpu.VMEM((CHUNK, D), jnp.float32),
        pltpu.VMEM((CHUNK, D), jnp.float32),
        pltpu.SemaphoreType.DMA,
        pltpu.SemaphoreType.DMA,
        pltpu.SemaphoreType.DMA,
    ]
    if with_cnt:
        out_type.append(jax.ShapeDtypeStruct((N_WORKERS, N_PAD), jnp.float32))
        scratch.append(pltpu.VMEM((N_PAD,), jnp.float32))

    def body(h_hbm, src_hbm, dst_hbm, z2d_hbm, z1d_hbm, acc_out, *rest):
        if with_cnt:
            (cnt_out, acc_sh, idx_s, idx_d, rows_a, rows_b,
             sem_a, sem_b, sem_i, cnt_local) = rest
        else:
            (acc_sh, idx_s, idx_d, rows_a, rows_b,
             sem_a, sem_b, sem_i) = rest
        cid = lax.axis_index("c")
        sid = lax.axis_index("s")
        wid = sid * 2 + cid
        gbase = wid * CHUNKS_PER_TILE
        r0 = sid * ROWS_PER_TILE

        def idx_load(g, slot):
            pltpu.async_copy(src_hbm.at[pl.ds(gbase + g * K, K)],
                             idx_s.at[slot], sem_i)
            pltpu.async_copy(dst_hbm.at[pl.ds(gbase + g * K, K)],
                             idx_d.at[slot], sem_i)

        def idx_wait(slot):
            pltpu.make_async_copy(src_hbm.at[pl.ds(0, K)],
                                  idx_s.at[slot], sem_i).wait()
            pltpu.make_async_copy(dst_hbm.at[pl.ds(0, K)],
                                  idx_d.at[slot], sem_i).wait()

        idx_load(0, 0)
        pltpu.sync_copy(z2d_hbm, acc_sh.at[pl.ds(r0, ROWS_PER_TILE)])
        if with_cnt:
            pltpu.sync_copy(z1d_hbm, cnt_local)

        idx_wait(0)
        pltpu.async_copy(h_hbm.at[idx_s.at[0, 0]], rows_a, sem_a)
        idx_load(1, 1)

        plsc.subcore_barrier()

        ones16 = jnp.ones((16,), jnp.float32)

        def run_group(g, slot, nslot):
            for c in range(K):
                cur, csem = (rows_a, sem_a) if c % 2 == 0 else (rows_b, sem_b)
                nxt, xsem = (rows_b, sem_b) if c % 2 == 0 else (rows_a, sem_a)
                if c < K - 1:
                    pltpu.async_copy(h_hbm.at[idx_s.at[slot, c + 1]],
                                     nxt, xsem)
                else:
                    @pl.when(g < G - 1)
                    def _():
                        idx_wait(nslot)
                        pltpu.async_copy(h_hbm.at[idx_s.at[nslot, 0]],
                                         nxt, xsem)
                pltpu.make_async_copy(h_hbm.at[idx_s.at[slot, c]],
                                      cur, csem).wait()
                pltpu.sync_copy(cur, acc_sh.at[idx_d.at[slot, c]], add=True)
                if with_cnt:
                    idx_row = idx_d.at[slot].at[c]
                    for i in range(CHUNK // 16):
                        idx16 = idx_row[pl.ds(i * 16, 16)]
                        plsc.addupdate_scatter(cnt_local, [idx16], ones16)

            @pl.when(g < G - 2)
            def _():
                idx_load(g + 2, slot)

        def pair_body(j, _):
            run_group(j * 2, 0, 1)
            run_group(j * 2 + 1, 1, 0)
            return 0
        lax.fori_loop(0, G // 2, pair_body, 0)

        if with_cnt:
            pltpu.sync_copy(cnt_local, cnt_out.at[wid])

        plsc.subcore_barrier()

        pltpu.sync_copy(acc_sh.at[pl.ds(r0, ROWS_PER_TILE)],
                        acc_out.at[cid].at[pl.ds(r0, ROWS_PER_TILE)])

    cparams = (pltpu.CompilerParams(needs_layout_passes=False)
               if with_cnt else None)
    return pl.kernel(body, out_type=out_type, mesh=mesh,
                     compiler_params=cparams, scratch_types=scratch)


_sc_agg_cnt = _make_sc_agg(True)
_sc_agg = _make_sc_agg(False)

_TC_ROWS = 1000


def _tc_layer_body(acc_ref, cnt_ref, h_ref, wl_ref, wr_ref, b_ref, out_ref):
    c = jnp.sum(cnt_ref[...], axis=0)
    s = acc_ref[0] + acc_ref[1]
    mean = s / jnp.maximum(c, 1.0)
    o = jnp.dot(mean, wl_ref[...], preferred_element_type=jnp.float32)
    o = o + jnp.dot(h_ref[...], wr_ref[...], preferred_element_type=jnp.float32)
    o = o + b_ref[...]
    out_ref[...] = jnp.maximum(o, 0.0)


def _tc_layer(acc, cnt3, h, W_l, W_r, b):
    grid = (N // _TC_ROWS,)
    return pl.pallas_call(
        _tc_layer_body,
        grid=grid,
        in_specs=[
            pl.BlockSpec((2, _TC_ROWS, D), lambda i: (0, i, 0)),
            pl.BlockSpec((N_WORKERS, _TC_ROWS, 1), lambda i: (0, i, 0)),
            pl.BlockSpec((_TC_ROWS, D), lambda i: (i, 0)),
            pl.BlockSpec((D, D), lambda i: (0, 0)),
            pl.BlockSpec((D, D), lambda i: (0, 0)),
            pl.BlockSpec((1, D), lambda i: (0, 0)),
        ],
        out_specs=pl.BlockSpec((_TC_ROWS, D), lambda i: (i, 0)),
        out_shape=jax.ShapeDtypeStruct((N, D), jnp.float32),
    )(acc, cnt3, h, W_l, W_r, b.reshape(1, D))


def kernel(x, edge_index, W1_l, b1, W1_r, W2_l, b2, W2_r):
    pad = E_PAD - E
    src_p = jnp.concatenate([edge_index[0],
                             jnp.zeros((pad,), jnp.int32)]).reshape(-1, CHUNK)
    dst_p = jnp.concatenate([edge_index[1],
                             jnp.full((pad,), DUMMY_ROW,
                                      jnp.int32)]).reshape(-1, CHUNK)
    z2d = jnp.zeros((ROWS_PER_TILE, D), jnp.float32)
    z1d = jnp.zeros((N_PAD,), jnp.float32)

    acc1, cnt = _sc_agg_cnt(x, src_p, dst_p, z2d, z1d)
    cnt3 = cnt.reshape(N_WORKERS, N_PAD, 1)
    h = _tc_layer(acc1, cnt3, x, W1_l, W1_r, b1)
    (acc2,) = _sc_agg(h, src_p, dst_p, z2d, z1d)
    out = _tc_layer(acc2, cnt3, h, W2_l, W2_r, b2)
    return out


# speedup vs baseline: 1.1689x; 1.1689x over previous
import jax
import jax.numpy as jnp
from jax import lax
from jax.experimental import pallas as pl
from jax.experimental.pallas import tpu as pltpu
from jax.experimental.pallas import tpu_sc as plsc

N = 10000
E = 320000
D = 128

N_PAD = 10240
DUMMY_ROW = N
CHUNK = 128
N_WORKERS = 32
K = 8
G0 = 16
G1 = 4
TOTAL_CHUNKS = 16 * K * (G0 + G1)
E_PAD = TOTAL_CHUNKS * CHUNK
ROWS_PER_TILE = N_PAD // 16


def _make_sc_agg(with_cnt: bool):
    mesh = plsc.VectorSubcoreMesh(core_axis_name="c", subcore_axis_name="s")
    out_type = [jax.ShapeDtypeStruct((2, N_PAD, D), jnp.float32)]
    scratch = [
        pltpu.VMEM_SHARED((N_PAD, D), jnp.float32),
        pltpu.VMEM((2, K, CHUNK), jnp.int32),
        pltpu.VMEM((2, K, CHUNK), jnp.int32),
        pltpu.VMEM((CHUNK, D), jnp.float32),
        pltpu.VMEM((CHUNK, D), jnp.float32),
        pltpu.SemaphoreType.DMA,
        pltpu.SemaphoreType.DMA,
        pltpu.SemaphoreType.DMA,
    ]
    if with_cnt:
        out_type.append(jax.ShapeDtypeStruct((N_WORKERS, N_PAD), jnp.float32))
        scratch.append(pltpu.VMEM((N_PAD,), jnp.float32))

    def body(h_hbm, src_hbm, dst_hbm, z2d_hbm, z1d_hbm, acc_out, *rest):
        if with_cnt:
            (cnt_out, acc_sh, idx_s, idx_d, rows_a, rows_b,
             sem_a, sem_b, sem_i, cnt_local) = rest
        else:
            (acc_sh, idx_s, idx_d, rows_a, rows_b,
             sem_a, sem_b, sem_i) = rest
        cid = lax.axis_index("c")
        sid = lax.axis_index("s")
        wid = sid * 2 + cid
        r0 = sid * ROWS_PER_TILE
        gbase = jnp.where(cid == 0, sid * (G0 * K),
                          16 * G0 * K + sid * (G1 * K))
        n_groups = jnp.where(cid == 0, G0, G1)

        def idx_load(g, slot):
            pltpu.async_copy(src_hbm.at[pl.ds(gbase + g * K, K)],
                             idx_s.at[slot], sem_i)
            pltpu.async_copy(dst_hbm.at[pl.ds(gbase + g * K, K)],
                             idx_d.at[slot], sem_i)

        def idx_wait(slot):
            pltpu.make_async_copy(src_hbm.at[pl.ds(0, K)],
                                  idx_s.at[slot], sem_i).wait()
            pltpu.make_async_copy(dst_hbm.at[pl.ds(0, K)],
                                  idx_d.at[slot], sem_i).wait()

        idx_load(0, 0)
        pltpu.sync_copy(z2d_hbm, acc_sh.at[pl.ds(r0, ROWS_PER_TILE)])
        if with_cnt:
            pltpu.sync_copy(z1d_hbm, cnt_local)

        idx_wait(0)
        pltpu.async_copy(h_hbm.at[idx_s.at[0, 0]], rows_a, sem_a)
        idx_load(1, 1)

        plsc.subcore_barrier()

        ones16 = jnp.ones((16,), jnp.float32)

        def run_group(g, slot, nslot):
            for c in range(K):
                cur, csem = (rows_a, sem_a) if c % 2 == 0 else (rows_b, sem_b)
                nxt, xsem = (rows_b, sem_b) if c % 2 == 0 else (rows_a, sem_a)
                if c < K - 1:
                    pltpu.async_copy(h_hbm.at[idx_s.at[slot, c + 1]],
                                     nxt, xsem)
                else:
                    @pl.when(g < n_groups - 1)
                    def _():
                        idx_wait(nslot)
                        pltpu.async_copy(h_hbm.at[idx_s.at[nslot, 0]],
                                         nxt, xsem)
                pltpu.make_async_copy(h_hbm.at[idx_s.at[slot, c]],
                                      cur, csem).wait()
                pltpu.sync_copy(cur, acc_sh.at[idx_d.at[slot, c]], add=True)
                if with_cnt:
                    idx_row = idx_d.at[slot].at[c]
                    for i in range(CHUNK // 16):
                        idx16 = idx_row[pl.ds(i * 16, 16)]
                        plsc.addupdate_scatter(cnt_local, [idx16], ones16)

            @pl.when(g < n_groups - 2)
            def _():
                idx_load(g + 2, slot)

        def pair_body(j, _):
            run_group(j * 2, 0, 1)
            run_group(j * 2 + 1, 1, 0)
            return 0
        lax.fori_loop(0, n_groups // 2, pair_body, 0)

        if with_cnt:
            pltpu.sync_copy(cnt_local, cnt_out.at[wid])

        plsc.subcore_barrier()

        pltpu.sync_copy(acc_sh.at[pl.ds(r0, ROWS_PER_TILE)],
                        acc_out.at[cid].at[pl.ds(r0, ROWS_PER_TILE)])

    cparams = (pltpu.CompilerParams(needs_layout_passes=False)
               if with_cnt else None)
    return pl.kernel(body, out_type=out_type, mesh=mesh,
                     compiler_params=cparams, scratch_types=scratch)


_sc_agg_cnt = _make_sc_agg(True)
_sc_agg = _make_sc_agg(False)

_TC_ROWS = 1024


def _tc_layer_body(acc_ref, cnt_ref, h_ref, wl_ref, wr_ref, b_ref, out_ref):
    c = jnp.sum(cnt_ref[...], axis=1, keepdims=True)
    s = acc_ref[0] + acc_ref[1]
    mean = s / jnp.maximum(c, 1.0)
    o = jnp.dot(mean, wl_ref[...], preferred_element_type=jnp.float32)
    o = o + jnp.dot(h_ref[...], wr_ref[...], preferred_element_type=jnp.float32)
    o = o + b_ref[...]
    out_ref[...] = jnp.maximum(o, 0.0)


def _tc_layer(acc, cnt_t, h, W_l, W_r, b):
    grid = (N_PAD // _TC_ROWS,)
    return pl.pallas_call(
        _tc_layer_body,
        grid=grid,
        in_specs=[
            pl.BlockSpec((2, _TC_ROWS, D), lambda i: (0, i, 0)),
            pl.BlockSpec((_TC_ROWS, N_WORKERS), lambda i: (i, 0)),
            pl.BlockSpec((_TC_ROWS, D), lambda i: (i, 0)),
            pl.BlockSpec((D, D), lambda i: (0, 0)),
            pl.BlockSpec((D, D), lambda i: (0, 0)),
            pl.BlockSpec((1, D), lambda i: (0, 0)),
        ],
        out_specs=pl.BlockSpec((_TC_ROWS, D), lambda i: (i, 0)),
        out_shape=jax.ShapeDtypeStruct((N_PAD, D), jnp.float32),
    )(acc, cnt_t, h, W_l, W_r, b.reshape(1, D))


def kernel(x, edge_index, W1_l, b1, W1_r, W2_l, b2, W2_r):
    pad = E_PAD - E
    src_p = jnp.concatenate([edge_index[0],
                             jnp.zeros((pad,), jnp.int32)]).reshape(-1, CHUNK)
    dst_p = jnp.concatenate([edge_index[1],
                             jnp.full((pad,), DUMMY_ROW,
                                      jnp.int32)]).reshape(-1, CHUNK)
    z2d = jnp.zeros((ROWS_PER_TILE, D), jnp.float32)
    z1d = jnp.zeros((N_PAD,), jnp.float32)
    x_p = jnp.pad(x, ((0, N_PAD - N), (0, 0)))

    acc1, cnt = _sc_agg_cnt(x, src_p, dst_p, z2d, z1d)
    cnt_t = cnt.T
    h = _tc_layer(acc1, cnt_t, x_p, W1_l, W1_r, b1)
    (acc2,) = _sc_agg(h, src_p, dst_p, z2d, z1d)
    out = _tc_layer(acc2, cnt_t, h, W2_l, W2_r, b2)
    return out[:N]


# speedup vs baseline: 1.1759x; 1.0059x over previous
import jax
import jax.numpy as jnp
from jax import lax
from jax.experimental import pallas as pl
from jax.experimental.pallas import tpu as pltpu
from jax.experimental.pallas import tpu_sc as plsc

N = 10000
E = 320000
D = 128

N_PAD = 10240
DUMMY_ROW = N
CHUNK = 128
N_WORKERS = 32
K = 8
G0 = 16
G1 = 4
TOTAL_CHUNKS = 16 * K * (G0 + G1)
E_PAD = TOTAL_CHUNKS * CHUNK
ROWS_PER_TILE = N_PAD // 16


def _make_sc_agg(with_cnt: bool):
    mesh = plsc.VectorSubcoreMesh(core_axis_name="c", subcore_axis_name="s")
    out_type = [jax.ShapeDtypeStruct((2, N_PAD, D), jnp.float32)]
    scratch = [
        pltpu.VMEM_SHARED((N_PAD, D), jnp.float32),
        pltpu.VMEM((2, K, CHUNK), jnp.int32),
        pltpu.VMEM((2, K, CHUNK), jnp.int32),
        pltpu.VMEM((CHUNK, D), jnp.float32),
        pltpu.VMEM((CHUNK, D), jnp.float32),
        pltpu.SemaphoreType.DMA,
        pltpu.SemaphoreType.DMA,
    ]
    if with_cnt:
        out_type.append(jax.ShapeDtypeStruct((N_WORKERS, N_PAD), jnp.float32))
        scratch.append(pltpu.VMEM((N_PAD,), jnp.float32))

    def body(h_hbm, src_hbm, dst_hbm, z2d_hbm, z1d_hbm, acc_out, *rest):
        if with_cnt:
            (cnt_out, acc_sh, idx_s, idx_d, rows_a, rows_b,
             sem_a, sem_b, cnt_local) = rest
        else:
            (acc_sh, idx_s, idx_d, rows_a, rows_b,
             sem_a, sem_b) = rest
        cid = lax.axis_index("c")
        sid = lax.axis_index("s")
        wid = sid * 2 + cid
        r0 = sid * ROWS_PER_TILE
        gbase = jnp.where(cid == 0, sid * (G0 * K),
                          16 * G0 * K + sid * (G1 * K))
        n_groups = jnp.where(cid == 0, G0, G1)

        def idx_load(g, slot):
            pltpu.sync_copy(src_hbm.at[pl.ds(gbase + g * K, K)],
                            idx_s.at[slot])
            pltpu.sync_copy(dst_hbm.at[pl.ds(gbase + g * K, K)],
                            idx_d.at[slot])

        pltpu.sync_copy(z2d_hbm, acc_sh.at[pl.ds(r0, ROWS_PER_TILE)])
        if with_cnt:
            pltpu.sync_copy(z1d_hbm, cnt_local)

        idx_load(0, 0)
        pltpu.async_copy(h_hbm.at[idx_s.at[0, 0]], rows_a, sem_a)

        plsc.subcore_barrier()

        ones16 = jnp.ones((16,), jnp.float32)

        def run_group(g, slot, nslot):
            @pl.when(g < n_groups - 1)
            def _():
                idx_load(g + 1, nslot)
            for c in range(K):
                cur, csem = (rows_a, sem_a) if c % 2 == 0 else (rows_b, sem_b)
                nxt, xsem = (rows_b, sem_b) if c % 2 == 0 else (rows_a, sem_a)
                if c < K - 1:
                    pltpu.async_copy(h_hbm.at[idx_s.at[slot, c + 1]],
                                     nxt, xsem)
                else:
                    @pl.when(g < n_groups - 1)
                    def _():
                        pltpu.async_copy(h_hbm.at[idx_s.at[nslot, 0]],
                                         nxt, xsem)
                pltpu.make_async_copy(h_hbm.at[idx_s.at[slot, c]],
                                      cur, csem).wait()
                pltpu.sync_copy(cur, acc_sh.at[idx_d.at[slot, c]], add=True)
                if with_cnt:
                    idx_row = idx_d.at[slot].at[c]
                    for i in range(CHUNK // 16):
                        idx16 = idx_row[pl.ds(i * 16, 16)]
                        plsc.addupdate_scatter(cnt_local, [idx16], ones16)

        def pair_body(j, _):
            run_group(j * 2, 0, 1)
            run_group(j * 2 + 1, 1, 0)
            return 0
        lax.fori_loop(0, n_groups // 2, pair_body, 0)

        if with_cnt:
            pltpu.sync_copy(cnt_local, cnt_out.at[wid])

        plsc.subcore_barrier()

        pltpu.sync_copy(acc_sh.at[pl.ds(r0, ROWS_PER_TILE)],
                        acc_out.at[cid].at[pl.ds(r0, ROWS_PER_TILE)])

    cparams = (pltpu.CompilerParams(needs_layout_passes=False)
               if with_cnt else None)
    return pl.kernel(body, out_type=out_type, mesh=mesh,
                     compiler_params=cparams, scratch_types=scratch)


_sc_agg_cnt = _make_sc_agg(True)
_sc_agg = _make_sc_agg(False)

_TC_ROWS = 1024


def _tc_layer_body(acc_ref, cnt_ref, h_ref, wl_ref, wr_ref, b_ref, out_ref):
    c = jnp.sum(cnt_ref[...], axis=1, keepdims=True)
    s = acc_ref[0] + acc_ref[1]
    mean = s / jnp.maximum(c, 1.0)
    o = jnp.dot(mean, wl_ref[...], preferred_element_type=jnp.float32)
    o = o + jnp.dot(h_ref[...], wr_ref[...], preferred_element_type=jnp.float32)
    o = o + b_ref[...]
    out_ref[...] = jnp.maximum(o, 0.0)


def _tc_layer(acc, cnt_t, h, W_l, W_r, b):
    grid = (N_PAD // _TC_ROWS,)
    return pl.pallas_call(
        _tc_layer_body,
        grid=grid,
        in_specs=[
            pl.BlockSpec((2, _TC_ROWS, D), lambda i: (0, i, 0)),
            pl.BlockSpec((_TC_ROWS, N_WORKERS), lambda i: (i, 0)),
            pl.BlockSpec((_TC_ROWS, D), lambda i: (i, 0)),
            pl.BlockSpec((D, D), lambda i: (0, 0)),
            pl.BlockSpec((D, D), lambda i: (0, 0)),
            pl.BlockSpec((1, D), lambda i: (0, 0)),
        ],
        out_specs=pl.BlockSpec((_TC_ROWS, D), lambda i: (i, 0)),
        out_shape=jax.ShapeDtypeStruct((N_PAD, D), jnp.float32),
    )(acc, cnt_t, h, W_l, W_r, b.reshape(1, D))


def kernel(x, edge_index, W1_l, b1, W1_r, W2_l, b2, W2_r):
    pad = E_PAD - E
    src_p = jnp.concatenate([edge_index[0],
                             jnp.zeros((pad,), jnp.int32)]).reshape(-1, CHUNK)
    dst_p = jnp.concatenate([edge_index[1],
                             jnp.full((pad,), DUMMY_ROW,
                                      jnp.int32)]).reshape(-1, CHUNK)
    z2d = jnp.zeros((ROWS_PER_TILE, D), jnp.float32)
    z1d = jnp.zeros((N_PAD,), jnp.float32)
    x_p = jnp.pad(x, ((0, N_PAD - N), (0, 0)))

    acc1, cnt = _sc_agg_cnt(x, src_p, dst_p, z2d, z1d)
    cnt_t = cnt.T
    h = _tc_layer(acc1, cnt_t, x_p, W1_l, W1_r, b1)
    (acc2,) = _sc_agg(h, src_p, dst_p, z2d, z1d)
    out = _tc_layer(acc2, cnt_t, h, W2_l, W2_r, b2)
    return out[:N]


# speedup vs baseline: 3.8289x; 3.2562x over previous
import jax
import jax.numpy as jnp
from jax import lax
from jax.experimental import pallas as pl
from jax.experimental.pallas import tpu as pltpu
from jax.experimental.pallas import tpu_sc as plsc

N = 10000
E = 320000
D = 128

N_PAD = 10240
DUMMY_ROW = N
CHUNK = 128
N_WORKERS = 32
K = 8
G0 = 10
G1 = 10
TOTAL_CHUNKS = 16 * K * (G0 + G1)
E_PAD = TOTAL_CHUNKS * CHUNK
ROWS_PER_TILE = N_PAD // 16


def _make_sc_agg(with_cnt: bool):
    mesh = plsc.VectorSubcoreMesh(core_axis_name="c", subcore_axis_name="s")
    out_type = [jax.ShapeDtypeStruct((2, N_PAD, D), jnp.float32)]
    scratch = [
        pltpu.VMEM_SHARED((N_PAD, D), jnp.float32),
        pltpu.VMEM((2, K, CHUNK), jnp.int32),
        pltpu.VMEM((2, K, CHUNK), jnp.int32),
        pltpu.VMEM((CHUNK, D), jnp.float32),
        pltpu.VMEM((CHUNK, D), jnp.float32),
        pltpu.SemaphoreType.DMA,
        pltpu.SemaphoreType.DMA,
    ]
    if with_cnt:
        out_type.append(jax.ShapeDtypeStruct((N_WORKERS, N_PAD), jnp.float32))
        scratch.append(pltpu.VMEM((N_PAD,), jnp.float32))

    def body(h_hbm, src_hbm, dst_hbm, z2d_hbm, z1d_hbm, acc_out, *rest):
        if with_cnt:
            (cnt_out, acc_sh, idx_s, idx_d, rows_a, rows_b,
             sem_a, sem_b, cnt_local) = rest
        else:
            (acc_sh, idx_s, idx_d, rows_a, rows_b,
             sem_a, sem_b) = rest
        cid = lax.axis_index("c")
        sid = lax.axis_index("s")
        wid = sid * 2 + cid
        r0 = sid * ROWS_PER_TILE
        gbase = jnp.where(cid == 0, sid * (G0 * K),
                          16 * G0 * K + sid * (G1 * K))
        n_groups = jnp.where(cid == 0, G0, G1)

        def idx_load(g, slot):
            pltpu.sync_copy(src_hbm.at[pl.ds(gbase + g * K, K)],
                            idx_s.at[slot])
            pltpu.sync_copy(dst_hbm.at[pl.ds(gbase + g * K, K)],
                            idx_d.at[slot])

        with jax.named_scope("sc_zero"):
            pltpu.sync_copy(z2d_hbm, acc_sh.at[pl.ds(r0, ROWS_PER_TILE)])
            if with_cnt:
                pltpu.sync_copy(z1d_hbm, cnt_local)

        with jax.named_scope("sc_prime"):
            idx_load(0, 0)
            pltpu.async_copy(h_hbm.at[idx_s.at[0, 0]], rows_a, sem_a)

        with jax.named_scope("sc_barrier1"):
            plsc.subcore_barrier()

        ones16 = jnp.ones((16,), jnp.float32)

        def run_group(g, slot, nslot):
            @pl.when(g < n_groups - 1)
            def _():
                idx_load(g + 1, nslot)
            for c in range(K):
                cur, csem = (rows_a, sem_a) if c % 2 == 0 else (rows_b, sem_b)
                nxt, xsem = (rows_b, sem_b) if c % 2 == 0 else (rows_a, sem_a)
                if c < K - 1:
                    pltpu.async_copy(h_hbm.at[idx_s.at[slot, c + 1]],
                                     nxt, xsem)
                else:
                    @pl.when(g < n_groups - 1)
                    def _():
                        pltpu.async_copy(h_hbm.at[idx_s.at[nslot, 0]],
                                         nxt, xsem)
                pltpu.make_async_copy(h_hbm.at[idx_s.at[slot, c]],
                                      cur, csem).wait()
                pltpu.sync_copy(cur, acc_sh.at[idx_d.at[slot, c]], add=True)
                if with_cnt:
                    idx_row = idx_d.at[slot].at[c]
                    for i in range(CHUNK // 16):
                        idx16 = idx_row[pl.ds(i * 16, 16)]
                        plsc.addupdate_scatter(cnt_local, [idx16], ones16)

        def pair_body(j, _):
            run_group(j * 2, 0, 1)
            run_group(j * 2 + 1, 1, 0)
            return 0
        with jax.named_scope("sc_mainloop"):
            lax.fori_loop(0, n_groups // 2, pair_body, 0)

        with jax.named_scope("sc_cntout"):
            if with_cnt:
                pltpu.sync_copy(cnt_local, cnt_out.at[wid])

        with jax.named_scope("sc_barrier2"):
            plsc.subcore_barrier()

        with jax.named_scope("sc_accout"):
            pltpu.sync_copy(acc_sh.at[pl.ds(r0, ROWS_PER_TILE)],
                            acc_out.at[cid].at[pl.ds(r0, ROWS_PER_TILE)])

    cparams = (pltpu.CompilerParams(needs_layout_passes=False)
               if with_cnt else None)
    return pl.kernel(body, out_type=out_type, mesh=mesh,
                     compiler_params=cparams, scratch_types=scratch)


_sc_agg_cnt = _make_sc_agg(True)
_sc_agg = _make_sc_agg(False)

_TC_ROWS = 1024


def _tc_layer_body(acc_ref, cnt_ref, h_ref, wl_ref, wr_ref, b_ref, out_ref):
    c = jnp.sum(cnt_ref[...], axis=1, keepdims=True)
    s = acc_ref[0] + acc_ref[1]
    mean = s / jnp.maximum(c, 1.0)
    o = jnp.dot(mean, wl_ref[...], preferred_element_type=jnp.float32)
    o = o + jnp.dot(h_ref[...], wr_ref[...], preferred_element_type=jnp.float32)
    o = o + b_ref[...]
    out_ref[...] = jnp.maximum(o, 0.0)


def _tc_layer(acc, cnt_t, h, W_l, W_r, b):
    grid = (N_PAD // _TC_ROWS,)
    return pl.pallas_call(
        _tc_layer_body,
        grid=grid,
        in_specs=[
            pl.BlockSpec((2, _TC_ROWS, D), lambda i: (0, i, 0)),
            pl.BlockSpec((_TC_ROWS, N_WORKERS), lambda i: (i, 0)),
            pl.BlockSpec((_TC_ROWS, D), lambda i: (i, 0)),
            pl.BlockSpec((D, D), lambda i: (0, 0)),
            pl.BlockSpec((D, D), lambda i: (0, 0)),
            pl.BlockSpec((1, D), lambda i: (0, 0)),
        ],
        out_specs=pl.BlockSpec((_TC_ROWS, D), lambda i: (i, 0)),
        out_shape=jax.ShapeDtypeStruct((N_PAD, D), jnp.float32),
    )(acc, cnt_t, h, W_l, W_r, b.reshape(1, D))


def kernel(x, edge_index, W1_l, b1, W1_r, W2_l, b2, W2_r):
    pad = E_PAD - E
    pad_src = jnp.arange(pad, dtype=jnp.int32) % N
    pad_dst = N + (jnp.arange(pad, dtype=jnp.int32) % (N_PAD - N))
    src_p = jnp.concatenate([edge_index[0], pad_src]).reshape(-1, CHUNK)
    dst_p = jnp.concatenate([edge_index[1], pad_dst]).reshape(-1, CHUNK)
    z2d = jnp.zeros((ROWS_PER_TILE, D), jnp.float32)
    z1d = jnp.zeros((N_PAD,), jnp.float32)
    x_p = jnp.pad(x, ((0, N_PAD - N), (0, 0)))

    acc1, cnt = _sc_agg_cnt(x, src_p, dst_p, z2d, z1d)
    cnt_t = cnt.T
    h = _tc_layer(acc1, cnt_t, x_p, W1_l, W1_r, b1)
    (acc2,) = _sc_agg(h, src_p, dst_p, z2d, z1d)
    out = _tc_layer(acc2, cnt_t, h, W2_l, W2_r, b2)
    return out[:N]
